# baseline (speedup 1.0000x reference)
import functools

import jax
import jax.numpy as jnp
from jax import lax
from jax.experimental import pallas as pl
from jax.experimental.pallas import tpu as pltpu
from jax.experimental.pallas import tpu_sc as plsc

_KERNEL = 2

_NC = 2
_NS = 16
_LANES = 16
_CHUNK = 128
_TOP_ROWS = 72


def _build_pool_kernel(num_planes, plane_sz, out_sz, out_w,
                       planes_per_worker):
  top_sz = _TOP_ROWS * out_w
  bot_sz = out_sz - top_sz
  stage_sz = _KERNEL * _TOP_ROWS * 2 * out_w
  n_vec_top = top_sz // _LANES
  n_vec_bot = bot_sz // _LANES
  n_chunk = bot_sz // _CHUNK
  assert top_sz % _CHUNK == 0 and bot_sz % _CHUNK == 0

  mesh = plsc.VectorSubcoreMesh(
      core_axis_name="c", subcore_axis_name="s", num_cores=_NC,
      num_subcores=_NS)

  @functools.partial(
      pl.kernel,
      out_type=jax.ShapeDtypeStruct((num_planes * out_sz,), jnp.float32),
      mesh=mesh,
      compiler_params=pltpu.CompilerParams(
          needs_layout_passes=False, use_tc_tiling_on_sc=False),
      scratch_types=[
          pltpu.VMEM((out_sz,), jnp.int32),
          pltpu.VMEM((bot_sz,), jnp.int32),
          pltpu.VMEM((bot_sz,), jnp.int32),
          pltpu.VMEM((stage_sz,), jnp.float32),
          pltpu.VMEM((stage_sz,), jnp.float32),
          pltpu.VMEM((out_sz,), jnp.float32),
          pltpu.VMEM((out_sz,), jnp.float32),
          pltpu.SemaphoreType.DMA,
          pltpu.SemaphoreType.DMA,
          pltpu.SemaphoreType.DMA,
          pltpu.SemaphoreType.DMA,
      ],
  )
  def pool_kernel(x_hbm, off_hbm, out_hbm, off_v, ig0, ig1, in0, in1, ob0,
                  ob1, sl0, sl1, si0, si1):
    c = lax.axis_index("c")
    s = lax.axis_index("s")
    wid = c * _NS + s
    base = wid * planes_per_worker
    pltpu.sync_copy(off_hbm.at[pl.ds(c * out_sz, out_sz)], off_v)

    idxgs = [ig0, ig1]
    ins = [in0, in1]
    obufs = [ob0, ob1]
    sems_lin = [sl0, sl1]
    sems_ind = [si0, si1]

    def addr_ind(k, b):
      pb = (base + k) * plane_sz

      def _addr(vi, carry):
        idxgs[b][pl.ds(vi * _LANES, _LANES)] = (
            off_v[pl.ds(top_sz + vi * _LANES, _LANES)] + pb)
        return carry

      lax.fori_loop(0, n_vec_bot, _addr, 0)

    def fire_lin(k, b):
      return pltpu.async_copy(
          x_hbm.at[pl.ds((base + k) * plane_sz, stage_sz)], ins[b],
          sems_lin[b])

    def fire_ind(b):
      def _fire(r, carry):
        pltpu.async_copy(
            x_hbm.at[idxgs[b].at[pl.ds(r * _CHUNK, _CHUNK)]],
            obufs[b].at[pl.ds(top_sz + r * _CHUNK, _CHUNK)], sems_ind[b])
        return carry

      lax.fori_loop(0, n_chunk, _fire, 0)

    def drain_ind(b):
      def _wait(r, carry):
        pltpu.make_async_copy(
            x_hbm.at[pl.ds(0, _CHUNK)],
            obufs[b].at[pl.ds(0, _CHUNK)], sems_ind[b]).wait()
        return carry

      lax.fori_loop(0, n_chunk, _wait, 0)

    addr_ind(0, 0)
    lin_copies = [None, None]
    lin_copies[0] = fire_lin(0, 0)
    fire_ind(0)
    if planes_per_worker > 1:
      addr_ind(1, 1)
    for k in range(planes_per_worker):
      b = k & 1
      nb = (k + 1) & 1
      if k + 1 < planes_per_worker:
        lin_copies[nb] = fire_lin(k + 1, nb)
        fire_ind(nb)
      lin_copies[b].wait()
      in_buf = ins[b]
      out_v = obufs[b]

      batch = 8

      def _gather(vi, carry):
        vbase = vi * (batch * _LANES)
        ivs = [off_v[pl.ds(vbase + u * _LANES, _LANES)]
               for u in range(batch)]
        vals = [plsc.load_gather(in_buf, [iv]) for iv in ivs]
        for u in range(batch):
          out_v[pl.ds(vbase + u * _LANES, _LANES)] = vals[u]
        return carry

      lax.fori_loop(0, n_vec_top // batch, _gather, 0)

      drain_ind(b)
      pltpu.sync_copy(out_v, out_hbm.at[pl.ds((base + k) * out_sz, out_sz)])
      if k + 2 < planes_per_worker:
        addr_ind(k + 2, b)

  return pool_kernel


def kernel(x, T):
  B, C, H, W = x.shape
  k = _KERNEL
  out_h, out_w = H // k, W // k
  num_patch = out_h * out_w
  t_static = 4
  n_groups = B // t_static

  idx_key = jax.random.fold_in(jax.random.key(0), 1)
  sel = jax.random.randint(idx_key, (n_groups, 1, num_patch), 0, k * k)
  sel = sel[:, 0, :] + (jnp.asarray(T, sel.dtype) - t_static)
  sel = jnp.clip(sel, 0, k * k - 1).astype(jnp.int32)

  pp = jnp.arange(num_patch, dtype=jnp.int32)
  pi = pp // out_w
  pj = pp % out_w
  dh = sel // k
  dw = sel % k
  off = ((k * pi + dh) * W + (k * pj + dw)).astype(jnp.int32)

  num_planes = B * C
  planes_per_worker = num_planes // (_NC * _NS)
  pool = _build_pool_kernel(num_planes, H * W, num_patch, out_w,
                            planes_per_worker)
  out_flat = pool(x.reshape(-1), off.reshape(-1))
  return out_flat.reshape(B, C, out_h, out_w)
